# Initial kernel scaffold; baseline (speedup 1.0000x reference)
import jax
import jax.numpy as jnp
from jax.experimental import pallas as pl


def kernel(x, edge_index, edge_attr, W_node, b_node, W_edge, b_edge, W_self, W_msg, b_out):
    raise NotImplementedError("write your pallas kernel here")


# speedup vs baseline: 2.3969x; 2.3969x over previous
import functools

import jax
import jax.numpy as jnp
from jax import lax
from jax.experimental import pallas as pl
from jax.experimental.pallas import tpu as pltpu
from jax.experimental.pallas import tpu_sc as plsc

NC = 2
NS = 16
NW = NC * NS
CHUNK = 80


def _linear_body(x_ref, w_ref, b_ref, o_ref):
    o_ref[...] = (
        jnp.dot(x_ref[...], w_ref[...], preferred_element_type=jnp.float32)
        + b_ref[...]
    )


def _linear(x, w, b, row_block):
    n, k = x.shape
    d = w.shape[1]
    grid = n // row_block
    return pl.pallas_call(
        _linear_body,
        grid=(grid,),
        in_specs=[
            pl.BlockSpec((row_block, k), lambda i: (i, 0)),
            pl.BlockSpec((k, d), lambda i: (0, 0)),
            pl.BlockSpec((1, d), lambda i: (0, 0)),
        ],
        out_specs=pl.BlockSpec((row_block, d), lambda i: (i, 0)),
        out_shape=jax.ShapeDtypeStruct((n, d), jnp.float32),
    )(x, w, b.reshape(1, d))


def _final_body(h_ref, p_ref, ws_ref, wm_ref, b_ref, o_ref):
    agg = p_ref[0] + p_ref[1]
    o_ref[...] = jnp.maximum(
        jnp.dot(h_ref[...], ws_ref[...], preferred_element_type=jnp.float32)
        + jnp.dot(agg, wm_ref[...], preferred_element_type=jnp.float32)
        + b_ref[...],
        0.0,
    )


def _final(h, partial, w_self, w_msg, b_out, row_block):
    n, d = h.shape
    grid = n // row_block
    return pl.pallas_call(
        _final_body,
        grid=(grid,),
        in_specs=[
            pl.BlockSpec((row_block, d), lambda i: (i, 0)),
            pl.BlockSpec((NC, row_block, d), lambda i: (0, i, 0)),
            pl.BlockSpec((d, d), lambda i: (0, 0)),
            pl.BlockSpec((d, d), lambda i: (0, 0)),
            pl.BlockSpec((1, d), lambda i: (0, 0)),
        ],
        out_specs=pl.BlockSpec((row_block, d), lambda i: (i, 0)),
        out_shape=jax.ShapeDtypeStruct((n, d), jnp.float32),
    )(h, partial, w_self, w_msg, b_out.reshape(1, d))


def _make_sc_aggregate(n_nodes, n_edges, d):
    epw = n_edges // NW
    nchunk = epw // CHUNK
    rows_per_tile = n_nodes // NS
    zrows = rows_per_tile // 5

    mesh = plsc.VectorSubcoreMesh(core_axis_name="c", subcore_axis_name="s")

    @functools.partial(
        pl.kernel,
        mesh=mesh,
        out_type=jax.ShapeDtypeStruct((NC, n_nodes, d), jnp.float32),
        scratch_types=[
            pltpu.VMEM((CHUNK,), jnp.int32),
            pltpu.VMEM((CHUNK,), jnp.int32),
            pltpu.VMEM((CHUNK, d), jnp.float32),
            pltpu.VMEM((CHUNK, d), jnp.float32),
            pltpu.VMEM((zrows, d), jnp.float32),
            pltpu.VMEM_SHARED((n_nodes, d), jnp.float32),
            pltpu.SemaphoreType.DMA,
        ],
    )
    def sc_aggregate(h_hbm, src_hbm, dst_hbm, e_hbm, out_hbm,
                     src_v, dst_v, e_v, rows_v, zero_v, agg_sh, sem):
        c = lax.axis_index("c")
        s = lax.axis_index("s")
        wid = c * NS + s

        def zbody(r, _):
            for j in range(d // 16):
                zero_v[r, pl.ds(j * 16, 16)] = jnp.zeros((16,), jnp.float32)
            return _
        lax.fori_loop(0, zrows, zbody, None)
        row0 = s * rows_per_tile
        for j in range(rows_per_tile // zrows):
            pltpu.sync_copy(zero_v, agg_sh.at[pl.ds(row0 + j * zrows, zrows)])
        plsc.subcore_barrier()

        def chunk_body(i, _):
            base = wid * epw + i * CHUNK
            pltpu.sync_copy(src_hbm.at[pl.ds(base, CHUNK)], src_v)
            pltpu.sync_copy(dst_hbm.at[pl.ds(base, CHUNK)], dst_v)
            pltpu.sync_copy(e_hbm.at[pl.ds(base, CHUNK)], e_v)
            pltpu.async_copy(h_hbm.at[src_v], rows_v, sem).wait()

            def rbody(r, _):
                for j in range(d // 16):
                    sl = pl.ds(j * 16, 16)
                    rows_v[r, sl] = jnp.maximum(rows_v[r, sl] + e_v[r, sl], 0.0)
                return _
            lax.fori_loop(0, CHUNK, rbody, None)

            pltpu.sync_copy(rows_v, agg_sh.at[dst_v], add=True)
            return _
        lax.fori_loop(0, nchunk, chunk_body, None)

        plsc.subcore_barrier()
        drain = (n_nodes // NS) // 8 * 8
        pltpu.sync_copy(
            agg_sh.at[pl.ds(s * drain, drain)],
            out_hbm.at[c, pl.ds(s * drain, drain)],
        )
        rem = n_nodes - NS * drain
        if rem:
            @pl.when(s == NS - 1)
            def _():
                pltpu.sync_copy(
                    agg_sh.at[pl.ds(NS * drain, rem)],
                    out_hbm.at[c, pl.ds(NS * drain, rem)],
                )

    return sc_aggregate


def kernel(x, edge_index, edge_attr, W_node, b_node, W_edge, b_edge,
           W_self, W_msg, b_out):
    n_nodes, d_feat = x.shape
    n_edges = edge_attr.shape[0]
    d = W_node.shape[1]

    src = edge_index[0].astype(jnp.int32)
    dst = edge_index[1].astype(jnp.int32)

    h = _linear(x, W_node, b_node, row_block=2000)
    e = _linear(edge_attr, W_edge, b_edge, row_block=3200)

    partial = _make_sc_aggregate(n_nodes, n_edges, d)(h, src, dst, e)

    return _final(h, partial, W_self, W_msg, b_out, row_block=2000)


# speedup vs baseline: 3.7295x; 1.5560x over previous
import functools

import jax
import jax.numpy as jnp
from jax import lax
from jax.experimental import pallas as pl
from jax.experimental.pallas import tpu as pltpu
from jax.experimental.pallas import tpu_sc as plsc

NC = 2
NS = 16
NW = NC * NS
CHUNK = 80


def _linear_body(x_ref, w_ref, b_ref, o_ref):
    o_ref[...] = (
        jnp.dot(x_ref[...], w_ref[...], preferred_element_type=jnp.float32)
        + b_ref[...]
    )


def _linear(x, w, b, row_block):
    n, k = x.shape
    d = w.shape[1]
    grid = n // row_block
    return pl.pallas_call(
        _linear_body,
        grid=(grid,),
        in_specs=[
            pl.BlockSpec((row_block, k), lambda i: (i, 0)),
            pl.BlockSpec((k, d), lambda i: (0, 0)),
            pl.BlockSpec((1, d), lambda i: (0, 0)),
        ],
        out_specs=pl.BlockSpec((row_block, d), lambda i: (i, 0)),
        out_shape=jax.ShapeDtypeStruct((n, d), jnp.float32),
    )(x, w, b.reshape(1, d))


def _final_body(h_ref, p_ref, ws_ref, wm_ref, b_ref, o_ref):
    agg = p_ref[0] + p_ref[1]
    o_ref[...] = jnp.maximum(
        jnp.dot(h_ref[...], ws_ref[...], preferred_element_type=jnp.float32)
        + jnp.dot(agg, wm_ref[...], preferred_element_type=jnp.float32)
        + b_ref[...],
        0.0,
    )


def _final(h, partial, w_self, w_msg, b_out, row_block):
    n, d = h.shape
    grid = n // row_block
    return pl.pallas_call(
        _final_body,
        grid=(grid,),
        in_specs=[
            pl.BlockSpec((row_block, d), lambda i: (i, 0)),
            pl.BlockSpec((NC, row_block, d), lambda i: (0, i, 0)),
            pl.BlockSpec((d, d), lambda i: (0, 0)),
            pl.BlockSpec((d, d), lambda i: (0, 0)),
            pl.BlockSpec((1, d), lambda i: (0, 0)),
        ],
        out_specs=pl.BlockSpec((row_block, d), lambda i: (i, 0)),
        out_shape=jax.ShapeDtypeStruct((n, d), jnp.float32),
    )(h, partial, w_self, w_msg, b_out.reshape(1, d))


def _make_sc_aggregate(n_nodes, n_edges, d):
    epw = n_edges // NW
    nchunk = epw // CHUNK
    rows_per_tile = n_nodes // NS
    assert nchunk % 2 == 1 and nchunk >= 3

    mesh = plsc.VectorSubcoreMesh(core_axis_name="c", subcore_axis_name="s")

    @functools.partial(
        pl.kernel,
        mesh=mesh,
        out_type=jax.ShapeDtypeStruct((NC, n_nodes, d), jnp.float32),
        scratch_types=[
            [pltpu.VMEM((CHUNK,), jnp.int32)] * 2,
            [pltpu.VMEM((CHUNK,), jnp.int32)] * 2,
            [pltpu.VMEM((CHUNK, d), jnp.float32)] * 2,
            [pltpu.VMEM((CHUNK, d), jnp.float32)] * 2,
            pltpu.VMEM_SHARED((n_nodes, d), jnp.float32),
            [pltpu.SemaphoreType.DMA] * 2,
            [pltpu.SemaphoreType.DMA] * 2,
            [pltpu.SemaphoreType.DMA] * 2,
            [pltpu.SemaphoreType.DMA] * 2,
            [pltpu.SemaphoreType.DMA] * 2,
        ],
    )
    def sc_aggregate(h_hbm, src_hbm, dst_hbm, e_hbm, out_hbm,
                     src_v, dst_v, e_v, rows_v, agg_sh,
                     sem_fs, sem_fd, sem_fe, sem_g, sem_s):
        c = lax.axis_index("c")
        s = lax.axis_index("s")
        wid = c * NS + s

        def zbody(r, _):
            for j in range(d // 16):
                e_v[0][r, pl.ds(j * 16, 16)] = jnp.zeros((16,), jnp.float32)
            return _
        lax.fori_loop(0, CHUNK, zbody, None)
        row0 = s * rows_per_tile
        nfull = rows_per_tile // CHUNK
        zrem = rows_per_tile - nfull * CHUNK
        for j in range(nfull):
            pltpu.sync_copy(e_v[0], agg_sh.at[pl.ds(row0 + j * CHUNK, CHUNK)])
        if zrem:
            pltpu.sync_copy(
                e_v[0].at[pl.ds(0, zrem)],
                agg_sh.at[pl.ds(row0 + nfull * CHUNK, zrem)],
            )
        plsc.subcore_barrier()

        def fetch_issue(j, p):
            base = wid * epw + j * CHUNK
            pltpu.async_copy(src_hbm.at[pl.ds(base, CHUNK)], src_v[p], sem_fs[p])
            pltpu.async_copy(dst_hbm.at[pl.ds(base, CHUNK)], dst_v[p], sem_fd[p])
            pltpu.async_copy(e_hbm.at[pl.ds(base, CHUNK)], e_v[p], sem_fe[p])

        def fetch_wait(j, p):
            base = wid * epw + j * CHUNK
            pltpu.make_async_copy(src_hbm.at[pl.ds(base, CHUNK)], src_v[p], sem_fs[p]).wait()
            pltpu.make_async_copy(dst_hbm.at[pl.ds(base, CHUNK)], dst_v[p], sem_fd[p]).wait()
            pltpu.make_async_copy(e_hbm.at[pl.ds(base, CHUNK)], e_v[p], sem_fe[p]).wait()

        def gather_issue(p):
            pltpu.async_copy(h_hbm.at[src_v[p]], rows_v[p], sem_g[p])

        def gather_wait(p):
            pltpu.make_async_copy(h_hbm.at[src_v[p]], rows_v[p], sem_g[p]).wait()

        def compute(p):
            def rbody(r, _):
                for j in range(d // 16):
                    sl = pl.ds(j * 16, 16)
                    rows_v[p][r, sl] = jnp.maximum(
                        rows_v[p][r, sl] + e_v[p][r, sl], 0.0)
                return _
            lax.fori_loop(0, CHUNK, rbody, None)

        def scatter_issue(p):
            pltpu.async_copy(rows_v[p], agg_sh.at[dst_v[p]], sem_s[p], add=True)

        def scatter_wait(p):
            pltpu.make_async_copy(rows_v[p], agg_sh.at[dst_v[p]], sem_s[p]).wait()

        fetch_issue(0, 0)
        fetch_wait(0, 0)
        gather_issue(0)
        fetch_issue(1, 1)
        gather_wait(0)
        compute(0)
        fetch_wait(1, 1)
        gather_issue(1)
        scatter_issue(0)

        def element(j, p):
            q = 1 - p
            scatter_wait(q)

            @pl.when(j < nchunk - 1)
            def _():
                fetch_issue(j + 1, q)
            gather_wait(p)
            compute(p)

            @pl.when(j < nchunk - 1)
            def _():
                fetch_wait(j + 1, q)
                gather_issue(q)
            scatter_issue(p)

        def pair(g, _):
            element(2 * g + 1, 1)
            element(2 * g + 2, 0)
            return _
        lax.fori_loop(0, (nchunk - 1) // 2, pair, None)
        scatter_wait((nchunk - 1) % 2)

        plsc.subcore_barrier()
        drain = (n_nodes // NS) // 8 * 8
        pltpu.sync_copy(
            agg_sh.at[pl.ds(s * drain, drain)],
            out_hbm.at[c, pl.ds(s * drain, drain)],
        )
        rem = n_nodes - NS * drain
        if rem:
            @pl.when(s == NS - 1)
            def _():
                pltpu.sync_copy(
                    agg_sh.at[pl.ds(NS * drain, rem)],
                    out_hbm.at[c, pl.ds(NS * drain, rem)],
                )

    return sc_aggregate


def kernel(x, edge_index, edge_attr, W_node, b_node, W_edge, b_edge,
           W_self, W_msg, b_out):
    n_nodes, d_feat = x.shape
    n_edges = edge_attr.shape[0]
    d = W_node.shape[1]

    src = edge_index[0].astype(jnp.int32)
    dst = edge_index[1].astype(jnp.int32)

    h = _linear(x, W_node, b_node, row_block=2000)
    e = _linear(edge_attr, W_edge, b_edge, row_block=3200)

    partial = _make_sc_aggregate(n_nodes, n_edges, d)(h, src, dst, e)

    return _final(h, partial, W_self, W_msg, b_out, row_block=2000)


# speedup vs baseline: 4.0159x; 1.0768x over previous
import functools

import jax
import jax.numpy as jnp
from jax import lax
from jax.experimental import pallas as pl
from jax.experimental.pallas import tpu as pltpu
from jax.experimental.pallas import tpu_sc as plsc

NC = 2
NS = 16
NW = NC * NS
CHUNK = 80


def _linear_body(x_ref, w_ref, b_ref, o_ref):
    o_ref[...] = (
        jnp.dot(x_ref[...], w_ref[...], preferred_element_type=jnp.float32)
        + b_ref[...]
    )


def _linear_body_bf16(x_ref, w_ref, b_ref, o_ref):
    o_ref[...] = (
        jnp.dot(x_ref[...].astype(jnp.bfloat16),
                w_ref[...].astype(jnp.bfloat16),
                preferred_element_type=jnp.float32)
        + b_ref[...]
    )


def _linear(x, w, b, row_block, bf16_mxu=False):
    n, k = x.shape
    d = w.shape[1]
    grid = n // row_block
    return pl.pallas_call(
        _linear_body_bf16 if bf16_mxu else _linear_body,
        grid=(grid,),
        in_specs=[
            pl.BlockSpec((row_block, k), lambda i: (i, 0)),
            pl.BlockSpec((k, d), lambda i: (0, 0)),
            pl.BlockSpec((1, d), lambda i: (0, 0)),
        ],
        out_specs=pl.BlockSpec((row_block, d), lambda i: (i, 0)),
        out_shape=jax.ShapeDtypeStruct((n, d), jnp.float32),
    )(x, w, b.reshape(1, d))


def _final_body(h_ref, p_ref, ws_ref, wm_ref, b_ref, o_ref):
    agg = p_ref[0] + p_ref[1]
    o_ref[...] = jnp.maximum(
        jnp.dot(h_ref[...], ws_ref[...], preferred_element_type=jnp.float32)
        + jnp.dot(agg, wm_ref[...], preferred_element_type=jnp.float32)
        + b_ref[...],
        0.0,
    )


def _final(h, partial, w_self, w_msg, b_out, row_block):
    n, d = h.shape
    grid = n // row_block
    return pl.pallas_call(
        _final_body,
        grid=(grid,),
        in_specs=[
            pl.BlockSpec((row_block, d), lambda i: (i, 0)),
            pl.BlockSpec((NC, row_block, d), lambda i: (0, i, 0)),
            pl.BlockSpec((d, d), lambda i: (0, 0)),
            pl.BlockSpec((d, d), lambda i: (0, 0)),
            pl.BlockSpec((1, d), lambda i: (0, 0)),
        ],
        out_specs=pl.BlockSpec((row_block, d), lambda i: (i, 0)),
        out_shape=jax.ShapeDtypeStruct((n, d), jnp.float32),
    )(h, partial, w_self, w_msg, b_out.reshape(1, d))


def _make_sc_aggregate(n_nodes, n_edges, d):
    epw = n_edges // NW
    nchunk = epw // CHUNK
    rows_per_tile = n_nodes // NS
    assert nchunk % 2 == 1 and nchunk >= 3

    mesh = plsc.VectorSubcoreMesh(core_axis_name="c", subcore_axis_name="s")

    @functools.partial(
        pl.kernel,
        mesh=mesh,
        out_type=jax.ShapeDtypeStruct((NC, n_nodes, d), jnp.float32),
        scratch_types=[
            [pltpu.VMEM((CHUNK,), jnp.int32)] * 2,
            [pltpu.VMEM((CHUNK,), jnp.int32)] * 2,
            [pltpu.VMEM((CHUNK, d), jnp.float32)] * 2,
            [pltpu.VMEM((CHUNK, d), jnp.float32)] * 2,
            pltpu.VMEM_SHARED((n_nodes, d), jnp.float32),
            [pltpu.SemaphoreType.DMA] * 2,
            [pltpu.SemaphoreType.DMA] * 2,
            [pltpu.SemaphoreType.DMA] * 2,
            [pltpu.SemaphoreType.DMA] * 2,
            [pltpu.SemaphoreType.DMA] * 2,
        ],
    )
    def sc_aggregate(h_hbm, ei_hbm, e_hbm, out_hbm,
                     src_v, dst_v, e_v, rows_v, agg_sh,
                     sem_fs, sem_fd, sem_fe, sem_g, sem_s):
        c = lax.axis_index("c")
        s = lax.axis_index("s")
        wid = c * NS + s

        def zbody(r, _):
            for j in range(d // 16):
                e_v[0][r, pl.ds(j * 16, 16)] = jnp.zeros((16,), jnp.float32)
            return _
        lax.fori_loop(0, CHUNK, zbody, None)
        row0 = s * rows_per_tile
        nfull = rows_per_tile // CHUNK
        zrem = rows_per_tile - nfull * CHUNK
        for j in range(nfull):
            pltpu.sync_copy(e_v[0], agg_sh.at[pl.ds(row0 + j * CHUNK, CHUNK)])
        if zrem:
            pltpu.sync_copy(
                e_v[0].at[pl.ds(0, zrem)],
                agg_sh.at[pl.ds(row0 + nfull * CHUNK, zrem)],
            )
        plsc.subcore_barrier()

        def fetch_issue(j, p):
            base = wid * epw + j * CHUNK
            pltpu.async_copy(ei_hbm.at[pl.ds(base, CHUNK)], src_v[p], sem_fs[p])
            pltpu.async_copy(ei_hbm.at[pl.ds(n_edges + base, CHUNK)], dst_v[p], sem_fd[p])
            pltpu.async_copy(e_hbm.at[pl.ds(base, CHUNK)], e_v[p], sem_fe[p])

        def fetch_wait(j, p):
            base = wid * epw + j * CHUNK
            pltpu.make_async_copy(ei_hbm.at[pl.ds(base, CHUNK)], src_v[p], sem_fs[p]).wait()
            pltpu.make_async_copy(ei_hbm.at[pl.ds(n_edges + base, CHUNK)], dst_v[p], sem_fd[p]).wait()
            pltpu.make_async_copy(e_hbm.at[pl.ds(base, CHUNK)], e_v[p], sem_fe[p]).wait()

        def gather_issue(p):
            pltpu.async_copy(h_hbm.at[src_v[p]], rows_v[p], sem_g[p])

        def gather_wait(p):
            pltpu.make_async_copy(h_hbm.at[src_v[p]], rows_v[p], sem_g[p]).wait()

        def compute(p):
            def rbody(r2, _):
                for u in range(2):
                    r = 2 * r2 + u
                    for j in range(d // 16):
                        sl = pl.ds(j * 16, 16)
                        rows_v[p][r, sl] = jnp.maximum(
                            rows_v[p][r, sl] + e_v[p][r, sl], 0.0)
                return _
            lax.fori_loop(0, CHUNK // 2, rbody, None)

        def scatter_issue(p):
            pltpu.async_copy(rows_v[p], agg_sh.at[dst_v[p]], sem_s[p], add=True)

        def scatter_wait(p):
            pltpu.make_async_copy(rows_v[p], agg_sh.at[dst_v[p]], sem_s[p]).wait()

        fetch_issue(0, 0)
        fetch_wait(0, 0)
        gather_issue(0)
        fetch_issue(1, 1)
        gather_wait(0)
        compute(0)
        fetch_wait(1, 1)
        gather_issue(1)
        scatter_issue(0)

        def element(j, p):
            q = 1 - p
            scatter_wait(q)

            @pl.when(j < nchunk - 1)
            def _():
                fetch_issue(j + 1, q)
            gather_wait(p)
            compute(p)

            @pl.when(j < nchunk - 1)
            def _():
                fetch_wait(j + 1, q)
                gather_issue(q)
            scatter_issue(p)

        def pair(g, _):
            element(2 * g + 1, 1)
            element(2 * g + 2, 0)
            return _
        lax.fori_loop(0, (nchunk - 1) // 2, pair, None)
        scatter_wait((nchunk - 1) % 2)

        plsc.subcore_barrier()
        drain = (n_nodes // NS) // 8 * 8
        pltpu.sync_copy(
            agg_sh.at[pl.ds(s * drain, drain)],
            out_hbm.at[c, pl.ds(s * drain, drain)],
        )
        rem = n_nodes - NS * drain
        if rem:
            @pl.when(s == NS - 1)
            def _():
                pltpu.sync_copy(
                    agg_sh.at[pl.ds(NS * drain, rem)],
                    out_hbm.at[c, pl.ds(NS * drain, rem)],
                )

    return sc_aggregate


def kernel(x, edge_index, edge_attr, W_node, b_node, W_edge, b_edge,
           W_self, W_msg, b_out):
    n_nodes, d_feat = x.shape
    n_edges = edge_attr.shape[0]
    d = W_node.shape[1]

    ei_flat = edge_index.astype(jnp.int32).reshape(-1)

    h = _linear(x, W_node, b_node, row_block=2000)
    e = _linear(edge_attr, W_edge, b_edge, row_block=6400, bf16_mxu=True)

    partial = _make_sc_aggregate(n_nodes, n_edges, d)(h, ei_flat, e)

    return _final(h, partial, W_self, W_msg, b_out, row_block=2000)


# speedup vs baseline: 4.9413x; 1.2304x over previous
import functools

import jax
import jax.numpy as jnp
from jax import lax
from jax.experimental import pallas as pl
from jax.experimental.pallas import tpu as pltpu
from jax.experimental.pallas import tpu_sc as plsc

NC = 2
NS = 16
NW = NC * NS
CHUNK = 80


def _node_body(x_ref, w_ref, b_ref, o_ref):
    o_ref[...] = (
        jnp.dot(x_ref[...], w_ref[...], preferred_element_type=jnp.float32)
        + b_ref[...]
    )


def _node_linear(x, w, b, row_block):
    n, k = x.shape
    d = w.shape[1]
    grid = n // row_block
    return pl.pallas_call(
        _node_body,
        grid=(grid,),
        in_specs=[
            pl.BlockSpec((row_block, k), lambda i: (i, 0)),
            pl.BlockSpec((k, d), lambda i: (0, 0)),
            pl.BlockSpec((1, d), lambda i: (0, 0)),
        ],
        out_specs=pl.BlockSpec((row_block, d), lambda i: (i, 0)),
        out_shape=jax.ShapeDtypeStruct((n, d), jnp.float32),
    )(x, w, b.reshape(1, d))


def _ei_split_body(ei_ref, src_ref, dst_ref):
    src_ref[...] = ei_ref[0, :]
    dst_ref[...] = ei_ref[1, :]


def _ei_split(ei):
    n_edges = ei.shape[1]
    return pl.pallas_call(
        _ei_split_body,
        out_shape=[
            jax.ShapeDtypeStruct((n_edges,), jnp.int32),
            jax.ShapeDtypeStruct((n_edges,), jnp.int32),
        ],
    )(ei)


def _edge_body(xt_ref, w_ref, b_ref, o_ref):
    o_ref[...] = (
        lax.dot_general(
            xt_ref[...].astype(jnp.bfloat16), w_ref[...].astype(jnp.bfloat16),
            dimension_numbers=(((0,), (0,)), ((), ())),
            preferred_element_type=jnp.float32)
        + b_ref[...]
    )


def _edge_linear(xt, w, b, col_block):
    k, n = xt.shape
    d = w.shape[1]
    return pl.pallas_call(
        _edge_body,
        grid=(n // col_block,),
        in_specs=[
            pl.BlockSpec((k, col_block), lambda i: (0, i)),
            pl.BlockSpec((k, d), lambda i: (0, 0)),
            pl.BlockSpec((1, d), lambda i: (0, 0)),
        ],
        out_specs=pl.BlockSpec((col_block, d), lambda i: (i, 0)),
        out_shape=jax.ShapeDtypeStruct((n, d), jnp.float32),
    )(xt, w, b.reshape(1, d))


def _final_body(h_ref, p_ref, ws_ref, wm_ref, b_ref, o_ref):
    agg = p_ref[0] + p_ref[1]
    o_ref[...] = jnp.maximum(
        jnp.dot(h_ref[...], ws_ref[...], preferred_element_type=jnp.float32)
        + jnp.dot(agg, wm_ref[...], preferred_element_type=jnp.float32)
        + b_ref[...],
        0.0,
    )


def _final(h, partial, w_self, w_msg, b_out, row_block):
    n, d = h.shape
    grid = n // row_block
    return pl.pallas_call(
        _final_body,
        grid=(grid,),
        in_specs=[
            pl.BlockSpec((row_block, d), lambda i: (i, 0)),
            pl.BlockSpec((NC, row_block, d), lambda i: (0, i, 0)),
            pl.BlockSpec((d, d), lambda i: (0, 0)),
            pl.BlockSpec((d, d), lambda i: (0, 0)),
            pl.BlockSpec((1, d), lambda i: (0, 0)),
        ],
        out_specs=pl.BlockSpec((row_block, d), lambda i: (i, 0)),
        out_shape=jax.ShapeDtypeStruct((n, d), jnp.float32),
    )(h, partial, w_self, w_msg, b_out.reshape(1, d))


def _make_sc_aggregate(n_nodes, n_edges, d):
    epw = n_edges // NW
    nchunk = epw // CHUNK
    rows_per_tile = n_nodes // NS
    assert nchunk % 2 == 1 and nchunk >= 3

    mesh = plsc.VectorSubcoreMesh(core_axis_name="c", subcore_axis_name="s")

    @functools.partial(
        pl.kernel,
        mesh=mesh,
        out_type=jax.ShapeDtypeStruct((NC, n_nodes, d), jnp.float32),
        scratch_types=[
            [pltpu.VMEM((CHUNK,), jnp.int32)] * 2,
            [pltpu.VMEM((CHUNK,), jnp.int32)] * 2,
            [pltpu.VMEM((CHUNK, d), jnp.float32)] * 2,
            [pltpu.VMEM((CHUNK, d), jnp.float32)] * 2,
            pltpu.VMEM_SHARED((n_nodes, d), jnp.float32),
            [pltpu.SemaphoreType.DMA] * 2,
            [pltpu.SemaphoreType.DMA] * 2,
            [pltpu.SemaphoreType.DMA] * 2,
            [pltpu.SemaphoreType.DMA] * 2,
            [pltpu.SemaphoreType.DMA] * 2,
        ],
    )
    def sc_aggregate(h_hbm, src_hbm, dst_hbm, e_hbm, out_hbm,
                     src_v, dst_v, e_v, rows_v, agg_sh,
                     sem_fs, sem_fd, sem_fe, sem_g, sem_s):
        c = lax.axis_index("c")
        s = lax.axis_index("s")
        wid = c * NS + s

        def zbody(r, _):
            for j in range(d // 16):
                e_v[0][r, pl.ds(j * 16, 16)] = jnp.zeros((16,), jnp.float32)
            return _
        lax.fori_loop(0, CHUNK, zbody, None)
        row0 = s * rows_per_tile
        nfull = rows_per_tile // CHUNK
        zrem = rows_per_tile - nfull * CHUNK
        for j in range(nfull):
            pltpu.sync_copy(e_v[0], agg_sh.at[pl.ds(row0 + j * CHUNK, CHUNK)])
        if zrem:
            pltpu.sync_copy(
                e_v[0].at[pl.ds(0, zrem)],
                agg_sh.at[pl.ds(row0 + nfull * CHUNK, zrem)],
            )
        plsc.subcore_barrier()

        def fetch_issue(j, p):
            base = wid * epw + j * CHUNK
            pltpu.async_copy(src_hbm.at[pl.ds(base, CHUNK)], src_v[p], sem_fs[p])
            pltpu.async_copy(dst_hbm.at[pl.ds(base, CHUNK)], dst_v[p], sem_fd[p])
            pltpu.async_copy(e_hbm.at[pl.ds(base, CHUNK)], e_v[p], sem_fe[p])

        def fetch_wait(j, p):
            base = wid * epw + j * CHUNK
            pltpu.make_async_copy(src_hbm.at[pl.ds(base, CHUNK)], src_v[p], sem_fs[p]).wait()
            pltpu.make_async_copy(dst_hbm.at[pl.ds(base, CHUNK)], dst_v[p], sem_fd[p]).wait()
            pltpu.make_async_copy(e_hbm.at[pl.ds(base, CHUNK)], e_v[p], sem_fe[p]).wait()

        def gather_issue(p):
            pltpu.async_copy(h_hbm.at[src_v[p]], rows_v[p], sem_g[p])

        def gather_wait(p):
            pltpu.make_async_copy(h_hbm.at[src_v[p]], rows_v[p], sem_g[p]).wait()

        def compute(p):
            def rbody(r2, _):
                for u in range(2):
                    r = 2 * r2 + u
                    for j in range(d // 16):
                        sl = pl.ds(j * 16, 16)
                        rows_v[p][r, sl] = jnp.maximum(
                            rows_v[p][r, sl] + e_v[p][r, sl], 0.0)
                return _
            lax.fori_loop(0, CHUNK // 2, rbody, None)

        def scatter_issue(p):
            pltpu.async_copy(rows_v[p], agg_sh.at[dst_v[p]], sem_s[p], add=True)

        def scatter_wait(p):
            pltpu.make_async_copy(rows_v[p], agg_sh.at[dst_v[p]], sem_s[p]).wait()

        fetch_issue(0, 0)
        fetch_wait(0, 0)
        gather_issue(0)
        fetch_issue(1, 1)
        gather_wait(0)
        compute(0)
        fetch_wait(1, 1)
        gather_issue(1)
        scatter_issue(0)

        def element(j, p):
            q = 1 - p
            scatter_wait(q)

            @pl.when(j < nchunk - 1)
            def _():
                fetch_issue(j + 1, q)
            gather_wait(p)
            compute(p)

            @pl.when(j < nchunk - 1)
            def _():
                fetch_wait(j + 1, q)
                gather_issue(q)
            scatter_issue(p)

        def pair(g, _):
            element(2 * g + 1, 1)
            element(2 * g + 2, 0)
            return _
        lax.fori_loop(0, (nchunk - 1) // 2, pair, None)
        scatter_wait((nchunk - 1) % 2)

        plsc.subcore_barrier()
        drain = (n_nodes // NS) // 8 * 8
        pltpu.sync_copy(
            agg_sh.at[pl.ds(s * drain, drain)],
            out_hbm.at[c, pl.ds(s * drain, drain)],
        )
        rem = n_nodes - NS * drain
        if rem:
            @pl.when(s == NS - 1)
            def _():
                pltpu.sync_copy(
                    agg_sh.at[pl.ds(NS * drain, rem)],
                    out_hbm.at[c, pl.ds(NS * drain, rem)],
                )

    return sc_aggregate


def kernel(x, edge_index, edge_attr, W_node, b_node, W_edge, b_edge,
           W_self, W_msg, b_out):
    n_nodes, d_feat = x.shape
    n_edges = edge_attr.shape[0]
    d = W_node.shape[1]

    ei32 = edge_index.astype(jnp.int32)

    h = _node_linear(x, W_node, b_node, row_block=2000)
    src, dst = _ei_split(ei32)
    e = _edge_linear(edge_attr.T, W_edge, b_edge, col_block=3200)

    partial = _make_sc_aggregate(n_nodes, n_edges, d)(h, src, dst, e)

    return _final(h, partial, W_self, W_msg, b_out, row_block=2000)
